# baseline (speedup 1.0000x reference)
import jax
import jax.numpy as jnp
from jax.experimental import pallas as pl
from jax.experimental.pallas import tpu as pltpu

_N = 4096
_D = 64
_BM = 256
_NB = _N // _BM
_NS = 4
_KS = _N // _NS
_NS_ID = 2
_KS_ID = _N // _NS_ID


def _l2n(x):
    n = jnp.sqrt(jnp.sum(x * x, axis=1, keepdims=True))
    return x / jnp.maximum(n, 1e-12)


def _lrelu(x):
    return jnp.where(x >= 0, x, 0.01 * x)


def _dot(a, b):
    return jnp.dot(a.astype(jnp.bfloat16), b.astype(jnp.bfloat16),
                   preferred_element_type=jnp.float32)


def _ksplit_dot(stream_refs, rhs, ks):
    t = _dot(stream_refs[0][...], rhs[0:ks])
    for s in range(1, len(stream_refs)):
        t = t + _dot(stream_refs[s][...], rhs[s * ks:(s + 1) * ks])
    return t


def _full_spec(h, w):
    return pl.BlockSpec((h, w), lambda i: (0, 0))


_PARAMS = pltpu.CompilerParams(dimension_semantics=("arbitrary",))


def _id_body(*refs):
    ui = refs[0:2 * _NS_ID]
    iu = refs[2 * _NS_ID:4 * _NS_ID]
    iemb, uemb, wcat, ue, ie, ou, oi = refs[4 * _NS_ID:]
    wc = wcat[...]
    ws = wc[0:64] + wc[64:128] + wc[128:192] + wc[192:256]
    eu = _dot(iemb[...], ws) * 0.5
    ei = _dot(uemb[...], ws) * 0.5
    um = 0.0
    im = 0.0
    for s in range(_NS_ID):
        lo, hi = s * _KS_ID, (s + 1) * _KS_ID
        um = um + _dot(ui[s][...] + ui[_NS_ID + s][...], eu[lo:hi])
        im = im + _dot(iu[s][...] + iu[_NS_ID + s][...], ei[lo:hi])
    ou[...] = ue[...] + 0.36 * _l2n(um)
    oi[...] = ie[...] + 0.36 * _l2n(im)


def _mega_body(*refs):
    f0 = refs[0:_NS]
    ui = refs[_NS:2 * _NS]
    iu = refs[2 * _NS:3 * _NS]
    (f1, ug0, ig0, w10, b10, w20, b20, w11, b11, w21, b21,
     ufin, ifin, r0, r1, r2, r3, i1s) = refs[3 * _NS:]
    i = pl.program_id(0)

    @pl.when(i < _NB)
    def _enc():
        rows = pl.ds(i * _BM, _BM)
        h0 = 0.0
        for s in range(_NS):
            h0 = h0 + _dot(f0[s][...], w10[s * _KS:(s + 1) * _KS])
        h0 = _lrelu(h0 + b10[...])
        r0[rows, 0:64] = _lrelu(_dot(h0, w20[...]) + b20[...])
        h1 = _lrelu(_dot(f1[...], w11[...]) + b11[...])
        r0[rows, 64:128] = _lrelu(_dot(h1, w21[...]) + b21[...])
        r0[rows, 128:192] = ig0[rows, :]

    @pl.when((i >= _NB) & (i < 2 * _NB))
    def _pass_a():
        rows = pl.ds((i - _NB) * _BM, _BM)
        r1[rows, :] = _ksplit_dot(ui, r0, _KS)

    @pl.when((i >= 2 * _NB) & (i < 3 * _NB))
    def _pass_b():
        rows = pl.ds((i - 2 * _NB) * _BM, _BM)
        t = _ksplit_dot(iu, r1, _KS)
        r2[rows, 0:128] = t[:, 0:128]
        s = t[:, 128:192]
        i1s[rows, :] = s
        s = s - jnp.max(s, axis=1, keepdims=True)
        e = jnp.exp(s)
        r2[rows, 128:192] = e / jnp.sum(e, axis=1, keepdims=True)

    @pl.when((i >= 3 * _NB) & (i < 4 * _NB))
    def _pass_c():
        rows = pl.ds((i - 3 * _NB) * _BM, _BM)
        t = _ksplit_dot(ui, r2, _KS)
        r3[rows, :] = t
        ufin[...] = (ug0[rows, :] + r1[rows, 128:192] + t[:, 128:192]) / 3.0 \
            + 0.02 * (_l2n(t[:, 0:64]) + _l2n(t[:, 64:128]))

    @pl.when(i >= 4 * _NB)
    def _pass_d():
        rows = pl.ds((i - 4 * _NB) * _BM, _BM)
        t = _ksplit_dot(iu, r3, _KS)
        ifin[...] = (ig0[rows, :] + i1s[rows, :] + t[:, 128:192]) / 3.0 \
            + 0.02 * (_l2n(t[:, 0:64]) + _l2n(t[:, 64:128]))


def kernel(ui_graph, iu_graph, mm_ui_graph_0, mm_ui_graph_1, mm_iu_graph_0,
           mm_iu_graph_1, mm_feats_0, mm_feats_1,
           enc0_W1, enc0_b1, enc0_W2, enc0_b2,
           enc1_W1, enc1_b1, enc1_W2, enc1_b2,
           user_emb, item_emb, w_q, w_k, w_cat):
    del w_q, w_k
    f32 = jnp.float32
    k1 = enc0_W1.shape[1]
    k2 = enc1_W1.shape[0]
    k3 = enc1_W1.shape[1]
    nb = _NB

    def _clip(x, lo, hi):
        return jnp.minimum(jnp.maximum(x, lo), hi)

    def _rs(bm, w):
        return pl.BlockSpec((bm, w), lambda i: (i, 0))

    def _id_spec(s):
        return pl.BlockSpec((_BM, _KS_ID), lambda i, s=s: (i, s))

    id_specs = ([_id_spec(s) for s in range(_NS_ID)] * 2
                + [_id_spec(s) for s in range(_NS_ID)] * 2)
    ug0, ig0 = pl.pallas_call(
        _id_body,
        grid=(_N // _BM,),
        in_specs=id_specs + [
            _full_spec(_N, _D), _full_spec(_N, _D), _full_spec(4 * _D, _D),
            _rs(_BM, _D), _rs(_BM, _D),
        ],
        out_specs=[_rs(_BM, _D), _rs(_BM, _D)],
        out_shape=[jax.ShapeDtypeStruct((_N, _D), f32)] * 2,
        compiler_params=_PARAMS,
    )(mm_ui_graph_0, mm_ui_graph_0, mm_ui_graph_1, mm_ui_graph_1,
      mm_iu_graph_0, mm_iu_graph_0, mm_iu_graph_1, mm_iu_graph_1,
      item_emb, user_emb, w_cat, user_emb, item_emb)

    w = 3 * _D

    def _f0_spec(s):
        return pl.BlockSpec((_BM, _KS),
                            lambda i, s=s: (_clip(i, 0, nb - 1), s))

    def _ui_spec(s):
        return pl.BlockSpec(
            (_BM, _KS),
            lambda i, s=s: (jnp.where(i < 3 * nb, _clip(i - nb, 0, nb - 1),
                                      _clip(i - 3 * nb, 0, nb - 1)), s))

    def _iu_spec(s):
        return pl.BlockSpec(
            (_BM, _KS),
            lambda i, s=s: (jnp.where(i < 4 * nb, _clip(i - 2 * nb, 0, nb - 1),
                                      _clip(i - 4 * nb, 0, nb - 1)), s))

    f1_spec = pl.BlockSpec((_BM, k2), lambda i: (_clip(i, 0, nb - 1), 0))
    ufin_spec = pl.BlockSpec((_BM, _D),
                             lambda i: (_clip(i - 3 * nb, 0, nb - 1), 0))
    ifin_spec = pl.BlockSpec((_BM, _D),
                             lambda i: (_clip(i - 4 * nb, 0, nb - 1), 0))

    u_final, i_final = pl.pallas_call(
        _mega_body,
        grid=(5 * nb,),
        in_specs=(
            [_f0_spec(s) for s in range(_NS)]
            + [_ui_spec(s) for s in range(_NS)]
            + [_iu_spec(s) for s in range(_NS)]
            + [f1_spec,
               _full_spec(_N, _D), _full_spec(_N, _D),
               _full_spec(_N, k1), _full_spec(1, k1),
               _full_spec(k1, _D), _full_spec(1, _D),
               _full_spec(k2, k3), _full_spec(1, k3),
               _full_spec(k3, _D), _full_spec(1, _D)]),
        out_specs=[ufin_spec, ifin_spec],
        out_shape=[jax.ShapeDtypeStruct((_N, _D), f32)] * 2,
        scratch_shapes=[
            pltpu.VMEM((_N, w), f32),
            pltpu.VMEM((_N, w), f32),
            pltpu.VMEM((_N, w), f32),
            pltpu.VMEM((_N, w), f32),
            pltpu.VMEM((_N, _D), f32),
        ],
        compiler_params=_PARAMS,
    )(*([mm_feats_0] * _NS + [ui_graph] * _NS + [iu_graph] * _NS
        + [mm_feats_1, ug0, ig0,
           enc0_W1, enc0_b1.reshape(1, -1), enc0_W2, enc0_b2.reshape(1, -1),
           enc1_W1, enc1_b1.reshape(1, -1), enc1_W2, enc1_b2.reshape(1, -1)]))

    return u_final, i_final


# speedup vs baseline: 1.0747x; 1.0747x over previous
import jax
import jax.numpy as jnp
from jax.experimental import pallas as pl
from jax.experimental.pallas import tpu as pltpu

_N = 4096
_D = 64
_BME = 256
_NBE = _N // _BME
_BMP = 512
_NBP = _N // _BMP
_PH = [_NBE, _NBE + _NBP, _NBE + 2 * _NBP, _NBE + 3 * _NBP, _NBE + 4 * _NBP]
_BM_ID = 256


def _l2n(x):
    n = jnp.sqrt(jnp.sum(x * x, axis=1, keepdims=True))
    return x / jnp.maximum(n, 1e-12)


def _lrelu(x):
    return jnp.where(x >= 0, x, 0.01 * x)


def _dot(a, b):
    return jnp.dot(a.astype(jnp.bfloat16), b.astype(jnp.bfloat16),
                   preferred_element_type=jnp.float32)


def _row_spec(bm, w):
    return pl.BlockSpec((bm, w), lambda i: (i, 0))


def _full_spec(h, w):
    return pl.BlockSpec((h, w), lambda i: (0, 0))


_PARAMS = pltpu.CompilerParams(dimension_semantics=("arbitrary",))


def _id_body(ui0, ui1, iu0, iu1, iemb, uemb, wcat, ue, ie, ou, oi):
    wc = wcat[...]
    ws = wc[0:64] + wc[64:128] + wc[128:192] + wc[192:256]
    eu = _dot(iemb[...], ws) * 0.5
    ei = _dot(uemb[...], ws) * 0.5
    um = _dot(ui0[...] + ui1[...], eu)
    im = _dot(iu0[...] + iu1[...], ei)
    ou[...] = ue[...] + 0.36 * _l2n(um)
    oi[...] = ie[...] + 0.36 * _l2n(im)


def _mega_body(f0, f1, ui, iu, ug0, ig0,
               w10, b10, w20, b20, w11, b11, w21, b21,
               ufin, ifin,
               r0, r1, r2, r3, i1s):
    i = pl.program_id(0)

    @pl.when(i < _PH[0])
    def _enc():
        k = i
        rows = pl.ds(k * _BME, _BME)
        h0 = _lrelu(_dot(f0[...], w10[...]) + b10[...])
        r0[rows, 0:64] = _lrelu(_dot(h0, w20[...]) + b20[...]).astype(jnp.bfloat16)
        h1 = _lrelu(_dot(f1[...], w11[...]) + b11[...])
        r0[rows, 64:128] = _lrelu(_dot(h1, w21[...]) + b21[...]).astype(jnp.bfloat16)
        r0[rows, 128:192] = ig0[rows, :].astype(jnp.bfloat16)

    @pl.when((i >= _PH[0]) & (i < _PH[1]))
    def _pass_a():
        k = i - _PH[0]
        rows = pl.ds(k * _BMP, _BMP)
        t = _dot(ui[...], r0[...])
        r1[rows, :] = t.astype(jnp.bfloat16)

    @pl.when((i >= _PH[1]) & (i < _PH[2]))
    def _pass_b():
        k = i - _PH[1]
        rows = pl.ds(k * _BMP, _BMP)
        t = _dot(iu[...], r1[...])
        r2[rows, 0:128] = t[:, 0:128].astype(jnp.bfloat16)
        s = t[:, 128:192]
        i1s[rows, :] = s.astype(jnp.bfloat16)
        s = s - jnp.max(s, axis=1, keepdims=True)
        e = jnp.exp(s)
        r2[rows, 128:192] = (e / jnp.sum(e, axis=1, keepdims=True)).astype(jnp.bfloat16)

    @pl.when((i >= _PH[2]) & (i < _PH[3]))
    def _pass_c():
        k = i - _PH[2]
        rows = pl.ds(k * _BMP, _BMP)
        t = _dot(ui[...], r2[...])
        r3[rows, :] = t.astype(jnp.bfloat16)
        u1 = r1[rows, 128:192].astype(jnp.float32)
        ufin[...] = (ug0[rows, :] + u1 + t[:, 128:192]) / 3.0 + \
            0.02 * (_l2n(t[:, 0:64]) + _l2n(t[:, 64:128]))

    @pl.when(i >= _PH[3])
    def _pass_d():
        k = i - _PH[3]
        rows = pl.ds(k * _BMP, _BMP)
        t = _dot(iu[...], r3[...])
        ifin[...] = (ig0[rows, :] + i1s[rows, :].astype(jnp.float32)
                     + t[:, 128:192]) / 3.0 + \
            0.02 * (_l2n(t[:, 0:64]) + _l2n(t[:, 64:128]))


def kernel(ui_graph, iu_graph, mm_ui_graph_0, mm_ui_graph_1, mm_iu_graph_0,
           mm_iu_graph_1, mm_feats_0, mm_feats_1,
           enc0_W1, enc0_b1, enc0_W2, enc0_b2,
           enc1_W1, enc1_b1, enc1_W2, enc1_b2,
           user_emb, item_emb, w_q, w_k, w_cat):
    del w_q, w_k
    f32 = jnp.float32
    k1 = enc0_W1.shape[1]
    k2 = enc1_W1.shape[0]
    k3 = enc1_W1.shape[1]

    n_blk_id = _N // _BM_ID
    ug0, ig0 = pl.pallas_call(
        _id_body,
        grid=(n_blk_id,),
        in_specs=[
            _row_spec(_BM_ID, _N), _row_spec(_BM_ID, _N),
            _row_spec(_BM_ID, _N), _row_spec(_BM_ID, _N),
            _full_spec(_N, _D), _full_spec(_N, _D),
            _full_spec(4 * _D, _D),
            _row_spec(_BM_ID, _D), _row_spec(_BM_ID, _D),
        ],
        out_specs=[_row_spec(_BM_ID, _D), _row_spec(_BM_ID, _D)],
        out_shape=[jax.ShapeDtypeStruct((_N, _D), f32)] * 2,
        compiler_params=pltpu.CompilerParams(
            dimension_semantics=("parallel",)),
    )(mm_ui_graph_0, mm_ui_graph_1, mm_iu_graph_0, mm_iu_graph_1,
      item_emb, user_emb, w_cat, user_emb, item_emb)

    w = 3 * _D

    def _clip(x, lo, hi):
        return jnp.minimum(jnp.maximum(x, lo), hi)

    f0_spec = pl.BlockSpec((_BME, _N), lambda i: (_clip(i, 0, _NBE - 1), 0))
    f1_spec = pl.BlockSpec((_BME, k2), lambda i: (_clip(i, 0, _NBE - 1), 0))
    ui_spec = pl.BlockSpec(
        (_BMP, _N),
        lambda i: (jnp.where(i < _PH[1],
                             _clip(i - _PH[0], 0, _NBP - 1),
                             _clip(i - _PH[2], 0, _NBP - 1)), 0))
    iu_spec = pl.BlockSpec(
        (_BMP, _N),
        lambda i: (jnp.where(i < _PH[2],
                             _clip(i - _PH[1], 0, _NBP - 1),
                             _clip(i - _PH[3], 0, _NBP - 1)), 0))
    ufin_spec = pl.BlockSpec((_BMP, _D),
                             lambda i: (_clip(i - _PH[2], 0, _NBP - 1), 0))
    ifin_spec = pl.BlockSpec((_BMP, _D),
                             lambda i: (_clip(i - _PH[3], 0, _NBP - 1), 0))

    u_final, i_final = pl.pallas_call(
        _mega_body,
        grid=(_PH[4],),
        in_specs=[
            f0_spec, f1_spec, ui_spec, iu_spec,
            _full_spec(_N, _D), _full_spec(_N, _D),
            _full_spec(_N, k1), _full_spec(1, k1),
            _full_spec(k1, _D), _full_spec(1, _D),
            _full_spec(k2, k3), _full_spec(1, k3),
            _full_spec(k3, _D), _full_spec(1, _D),
        ],
        out_specs=[ufin_spec, ifin_spec],
        out_shape=[jax.ShapeDtypeStruct((_N, _D), f32)] * 2,
        scratch_shapes=[
            pltpu.VMEM((_N, w), jnp.bfloat16),
            pltpu.VMEM((_N, w), jnp.bfloat16),
            pltpu.VMEM((_N, w), jnp.bfloat16),
            pltpu.VMEM((_N, w), jnp.bfloat16),
            pltpu.VMEM((_N, _D), jnp.bfloat16),
        ],
        compiler_params=_PARAMS,
    )(mm_feats_0, mm_feats_1, ui_graph, iu_graph, ug0, ig0,
      enc0_W1, enc0_b1.reshape(1, -1), enc0_W2, enc0_b2.reshape(1, -1),
      enc1_W1, enc1_b1.reshape(1, -1), enc1_W2, enc1_b2.reshape(1, -1))

    return u_final, i_final
